# baseline (speedup 1.0000x reference)
import jax
import jax.numpy as jnp
from jax import lax
from jax.experimental import pallas as pl
from jax.experimental.pallas import tpu as pltpu
from jax.experimental.pallas import tpu_sc as plsc

CROP_H = 448
CROP_W = 448
ROW_BLK = 32
IMG_BLK = 96

N_IMG = 192
TOP = 32
LEFT = 32
NUM_WORKERS = 32
IMGS_PER_WORKER = N_IMG // NUM_WORKERS
ROW_CHUNK = 64
CHUNKS_PER_IMG = CROP_H // ROW_CHUNK
NBUF = 4


def _crop_body(x_ref, o_ref):
    o_ref[...] = x_ref[:, :, LEFT : LEFT + CROP_W]


def _tc_crop(xf, n_img, H, W):
    return pl.pallas_call(
        _crop_body,
        grid=(n_img // IMG_BLK, CROP_H // ROW_BLK),
        in_specs=[
            pl.BlockSpec(
                (IMG_BLK, ROW_BLK, W), lambda i, j: (i, j + TOP // ROW_BLK, 0)
            )
        ],
        out_specs=pl.BlockSpec((IMG_BLK, ROW_BLK, CROP_W), lambda i, j: (i, j, 0)),
        out_shape=jax.ShapeDtypeStruct((n_img, CROP_H, CROP_W), xf.dtype),
    )(xf)


def _sc_crop_body(x_hbm, idx_hbm, out_hbm, *scratch):
    bufs = scratch[:NBUF]
    idx_buf = scratch[NBUF]
    sems = scratch[NBUF + 1 :]
    c = lax.axis_index("c")
    s = lax.axis_index("s")
    wid = s * 2 + c

    tasks = []
    for k in range(IMGS_PER_WORKER):
        img = wid * IMGS_PER_WORKER + k
        for j in range(CHUNKS_PER_IMG):
            tasks.append((img, j * ROW_CHUNK))
    n = len(tasks)

    pltpu.sync_copy(idx_hbm.at[wid], idx_buf)

    out_h = {}
    for i in range(n):
        img, r0 = tasks[i]
        b = i % NBUF
        if i - NBUF >= 0:
            out_h[i - NBUF].wait()
        pltpu.async_copy(
            x_hbm.at[img, pl.ds(TOP + r0, ROW_CHUNK), pl.ds(LEFT, CROP_W)],
            bufs[b],
            sems[b],
        ).wait()
        out_h[i] = pltpu.async_copy(bufs[b], out_hbm.at[idx_buf.at[i]], sems[b])
    for i in range(max(0, n - NBUF), n):
        out_h[i].wait()


def _sc_crop(xf):
    n_rows = N_IMG * CROP_H
    idx = jnp.arange(n_rows, dtype=jnp.int32).reshape(
        NUM_WORKERS, IMGS_PER_WORKER * CHUNKS_PER_IMG, ROW_CHUNK
    )
    mesh = plsc.VectorSubcoreMesh(core_axis_name="c", subcore_axis_name="s")
    kfn = pl.kernel(
        _sc_crop_body,
        mesh=mesh,
        out_type=jax.ShapeDtypeStruct((n_rows, CROP_W), jnp.float32),
        scratch_types=(
            [pltpu.VMEM((ROW_CHUNK, CROP_W), jnp.float32) for _ in range(NBUF)]
            + [
                pltpu.VMEM(
                    (IMGS_PER_WORKER * CHUNKS_PER_IMG, ROW_CHUNK), jnp.int32
                )
            ]
            + [pltpu.SemaphoreType.DMA for _ in range(NBUF)]
        ),
        compiler_params=pltpu.CompilerParams(use_tc_tiling_on_sc=False),
    )
    return kfn(xf, idx)


def kernel(x):
    B, T, C, H, W = x.shape
    N = B * T * C
    xf = x.reshape(N, H, W)
    out = _sc_crop(xf)
    return out.reshape(B, T * C, CROP_H, CROP_W)


# speedup vs baseline: 1.0068x; 1.0068x over previous
import jax
import jax.numpy as jnp
from jax import lax
from jax.experimental import pallas as pl
from jax.experimental.pallas import tpu as pltpu
from jax.experimental.pallas import tpu_sc as plsc

CROP_H = 448
CROP_W = 448
ROW_BLK = 32
IMG_BLK = 96

N_IMG = 192
TOP = 32
LEFT = 32
NUM_WORKERS = 32
IMGS_PER_WORKER = N_IMG // NUM_WORKERS
ROW_CHUNK = 64
CHUNKS_PER_IMG = CROP_H // ROW_CHUNK
NBUF = 4


def _crop_body(x_ref, o_ref):
    o_ref[...] = x_ref[:, :, LEFT : LEFT + CROP_W]


def _tc_crop(xf, n_img, H, W):
    return pl.pallas_call(
        _crop_body,
        grid=(n_img // IMG_BLK, CROP_H // ROW_BLK),
        in_specs=[
            pl.BlockSpec(
                (IMG_BLK, ROW_BLK, W), lambda i, j: (i, j + TOP // ROW_BLK, 0)
            )
        ],
        out_specs=pl.BlockSpec((IMG_BLK, ROW_BLK, CROP_W), lambda i, j: (i, j, 0)),
        out_shape=jax.ShapeDtypeStruct((n_img, CROP_H, CROP_W), xf.dtype),
    )(xf)


def _sc_crop_body(x_hbm, idx_hbm, out_hbm, *scratch):
    bufs = scratch[:NBUF]
    idx_buf = scratch[NBUF]
    sin = scratch[NBUF + 1 : 2 * NBUF + 1]
    sout = scratch[2 * NBUF + 1 :]
    c = lax.axis_index("c")
    s = lax.axis_index("s")
    wid = s * 2 + c

    tasks = []
    for k in range(IMGS_PER_WORKER):
        img = wid * IMGS_PER_WORKER + k
        for j in range(CHUNKS_PER_IMG):
            tasks.append((img, j * ROW_CHUNK))
    n = len(tasks)

    pltpu.sync_copy(idx_hbm.at[wid], idx_buf)

    in_h = {}
    out_h = {}

    def start_in(m):
        img, r0 = tasks[m]
        b = m % NBUF
        in_h[m] = pltpu.async_copy(
            x_hbm.at[img, pl.ds(TOP + r0, ROW_CHUNK), pl.ds(LEFT, CROP_W)],
            bufs[b],
            sin[b],
        )

    start_in(0)
    start_in(1)
    for i in range(n):
        b = i % NBUF
        in_h[i].wait()
        out_h[i] = pltpu.async_copy(bufs[b], out_hbm.at[idx_buf.at[i]], sout[b])
        m = i + 2
        if m < n:
            if m - NBUF >= 0:
                out_h[m - NBUF].wait()
            start_in(m)
    for i in range(max(0, n - NBUF), n):
        out_h[i].wait()


def _sc_crop(xf):
    n_rows = N_IMG * CROP_H
    idx = jnp.arange(n_rows, dtype=jnp.int32).reshape(
        NUM_WORKERS, IMGS_PER_WORKER * CHUNKS_PER_IMG, ROW_CHUNK
    )
    mesh = plsc.VectorSubcoreMesh(core_axis_name="c", subcore_axis_name="s")
    kfn = pl.kernel(
        _sc_crop_body,
        mesh=mesh,
        out_type=jax.ShapeDtypeStruct((n_rows, CROP_W), jnp.float32),
        scratch_types=(
            [pltpu.VMEM((ROW_CHUNK, CROP_W), jnp.float32) for _ in range(NBUF)]
            + [
                pltpu.VMEM(
                    (IMGS_PER_WORKER * CHUNKS_PER_IMG, ROW_CHUNK), jnp.int32
                )
            ]
            + [pltpu.SemaphoreType.DMA for _ in range(2 * NBUF)]
        ),
        compiler_params=pltpu.CompilerParams(use_tc_tiling_on_sc=False),
    )
    return kfn(xf, idx)


def kernel(x):
    B, T, C, H, W = x.shape
    N = B * T * C
    xf = x.reshape(N, H, W)
    out = _sc_crop(xf)
    return out.reshape(B, T * C, CROP_H, CROP_W)
